# baseline (speedup 1.0000x reference)
import jax
import jax.numpy as jnp
from jax.experimental import pallas as pl
from jax.experimental.pallas import tpu as pltpu
from jax.experimental.shard_map import shard_map
from jax.sharding import Mesh, PartitionSpec as P

BS_ = 8
NC_ = 16
C_ = 128
D_ = 64
K_ = 16
KB_ = 8
BB_ = 16
H_ = 32
MODIN_ = K_ + 3 * D_ + 1
MODOUT_ = K_ + KB_ + 1 + D_
N_ = NC_ * C_
G_ = 8
NB_ = 256
CB_ = NB_ // C_
NG_ = NB_ // G_


def _block(bidx_ref, ht_ref, h_ref, dl_ref, pr_ref, nid_ref,
           w1_ref, b1_ref, w2_ref, b2_ref,
           wconn_ref, border_ref, decay_ref, prim_ref,
           xs_ref, bsc_ref):
    i = pl.program_id(0)
    rows = G_ * BS_

    xs_ref[:, :, 0:K_] = ht_ref[...]
    xs_ref[:, :, K_:K_ + D_] = h_ref[...]
    xs_ref[:, :, K_ + D_:K_ + D_ + 1] = dl_ref[...][..., None]
    xs_ref[:, :, K_ + D_ + 1:K_ + 2 * D_ + 1] = pr_ref[...]
    xs_ref[:, :, K_ + 2 * D_ + 1:] = jnp.broadcast_to(
        nid_ref[...][None], (BS_, NB_, D_))

    row_n = jax.lax.broadcasted_iota(jnp.int32, (rows, G_ * H_), 0) % G_
    col_n = jax.lax.broadcasted_iota(jnp.int32, (rows, G_ * H_), 1) // H_
    mask = (row_n == col_n).astype(jnp.float32)

    for g in range(NG_):
        sl = slice(g * G_, (g + 1) * G_)
        xg = xs_ref[:, sl, :].reshape(rows, MODIN_).astype(jnp.bfloat16)
        w1r = w1_ref[sl].reshape(G_ * H_, MODIN_).astype(jnp.bfloat16)
        acc = jax.lax.dot_general(
            xg, w1r, (((1,), (1,)), ((), ())),
            preferred_element_type=jnp.float32)
        b1w = b1_ref[:, g, :]
        hidw = (jnp.tanh(acc + b1w) * mask).astype(jnp.bfloat16)
        w2r = w2_ref[sl].reshape(G_ * H_, MODOUT_).astype(jnp.bfloat16)
        outg = jax.lax.dot_general(
            hidw, w2r, (((1,), (0,)), ((), ())),
            preferred_element_type=jnp.float32)
        outg = outg.reshape(BS_, G_, MODOUT_) + b2_ref[sl][None]
        wconn_ref[:, sl, :] = outg[..., :K_]
        bsc_ref[:, sl, :] = outg[..., K_:K_ + KB_]
        decay_ref[:, sl] = outg[..., K_ + KB_]
        prim_ref[:, sl, :] = outg[..., K_ + KB_ + 1:]

    for c in range(CB_):
        for j in range(BB_):
            idx = bidx_ref[i * CB_ + c, j]
            border_ref[:, c, j:j + 1, :] = bsc_ref[:, pl.ds(c * C_ + idx, 1), :]


def _local_mlp(bidx, htf, hf, dlf, prf, nidf, w1, b1, w2, b2):
    bs = hf.shape[0]
    n_loc = hf.shape[1]
    grid = n_loc // NB_
    nc_loc = n_loc // C_

    grid_spec = pltpu.PrefetchScalarGridSpec(
        num_scalar_prefetch=1,
        grid=(grid,),
        in_specs=[
            pl.BlockSpec((BS_, NB_, K_), lambda i, b: (0, i, 0)),
            pl.BlockSpec((BS_, NB_, D_), lambda i, b: (0, i, 0)),
            pl.BlockSpec((BS_, NB_), lambda i, b: (0, i)),
            pl.BlockSpec((BS_, NB_, D_), lambda i, b: (0, i, 0)),
            pl.BlockSpec((NB_, D_), lambda i, b: (i, 0)),
            pl.BlockSpec((NB_, H_, MODIN_), lambda i, b: (i, 0, 0)),
            pl.BlockSpec((1, NG_, G_ * H_), lambda i, b: (i, 0, 0)),
            pl.BlockSpec((NB_, H_, MODOUT_), lambda i, b: (i, 0, 0)),
            pl.BlockSpec((NB_, MODOUT_), lambda i, b: (i, 0)),
        ],
        out_specs=[
            pl.BlockSpec((BS_, NB_, K_), lambda i, b: (0, i, 0)),
            pl.BlockSpec((BS_, CB_, BB_, KB_), lambda i, b: (0, i, 0, 0)),
            pl.BlockSpec((BS_, NB_), lambda i, b: (0, i)),
            pl.BlockSpec((BS_, NB_, D_), lambda i, b: (0, i, 0)),
        ],
        scratch_shapes=[
            pltpu.VMEM((BS_, NB_, MODIN_), jnp.float32),
            pltpu.VMEM((BS_, NB_, KB_), jnp.float32),
        ],
    )
    return pl.pallas_call(
        _block,
        grid_spec=grid_spec,
        compiler_params=pltpu.CompilerParams(
            dimension_semantics=("arbitrary",)),
        out_shape=[
            jax.ShapeDtypeStruct((bs, n_loc, K_), jnp.float32),
            jax.ShapeDtypeStruct((bs, nc_loc, BB_, KB_), jnp.float32),
            jax.ShapeDtypeStruct((bs, n_loc), jnp.float32),
            jax.ShapeDtypeStruct((bs, n_loc, D_), jnp.float32),
        ],
    )(bidx, htf, hf, dlf, prf, nidf, w1,
      b1.reshape(grid, NG_, G_ * H_), w2, b2)


def kernel(h, hebbian_traces, decay_logit, primitives, mod_w1, mod_b1,
           mod_w2, mod_b2, neuron_id, border_indices):
    bs = h.shape[0]
    htf = hebbian_traces.reshape(bs, N_, K_)
    hf = h.reshape(bs, N_, D_)
    dlf = decay_logit.reshape(bs, N_)
    prf = primitives.reshape(bs, N_, D_)
    nidf = neuron_id.reshape(N_, D_)
    bidx = border_indices.astype(jnp.int32)

    devs = jax.devices()
    ndev = 2 if len(devs) >= 2 else 1
    if ndev > 1:
        mesh = Mesh(devs[:ndev], ("x",))
        sh = P(None, "x")
        wsh = P("x")
        fn = shard_map(
            _local_mlp, mesh=mesh,
            in_specs=(wsh, sh, sh, sh, sh, wsh, wsh, wsh, wsh, wsh),
            out_specs=(sh, sh, sh, sh),
            check_rep=False)
    else:
        fn = _local_mlp
    wconn, border, decay, prim = fn(
        bidx, htf, hf, dlf, prf, nidf, mod_w1, mod_b1, mod_w2, mod_b2)

    return (wconn.reshape(bs, NC_, C_, K_),
            border,
            decay.reshape(bs, NC_, C_),
            prim.reshape(bs, NC_, C_, D_))


# speedup vs baseline: 1.9876x; 1.9876x over previous
import jax
import jax.numpy as jnp
from jax.experimental import pallas as pl
from jax.experimental.pallas import tpu as pltpu
BS_ = 8
NC_ = 16
C_ = 128
D_ = 64
K_ = 16
KB_ = 8
BB_ = 16
H_ = 32
MODIN_ = K_ + 3 * D_ + 1
MODOUT_ = K_ + KB_ + 1 + D_
N_ = NC_ * C_
G_ = 32
NB_ = 256
CB_ = NB_ // C_
NG_ = NB_ // G_


def _block(bidx_ref, ht_ref, h_ref, dl_ref, pr_ref, nid_ref,
           w1_ref, b1_ref, w2_ref, b2_ref,
           wconn_ref, border_ref, decay_ref, prim_ref,
           xs_ref, bsc_ref):
    i = pl.program_id(0)
    rows = G_ * BS_

    xs_ref[:, :, 0:K_] = ht_ref[...].astype(jnp.bfloat16)
    xs_ref[:, :, K_:K_ + D_] = h_ref[...].astype(jnp.bfloat16)
    xs_ref[:, :, K_ + D_:K_ + D_ + 1] = dl_ref[...][..., None].astype(jnp.bfloat16)
    xs_ref[:, :, K_ + D_ + 1:K_ + 2 * D_ + 1] = pr_ref[...].astype(jnp.bfloat16)
    xs_ref[:, :, K_ + 2 * D_ + 1:] = jnp.broadcast_to(
        nid_ref[...].astype(jnp.bfloat16)[None], (BS_, NB_, D_))

    row_n = jax.lax.broadcasted_iota(jnp.int32, (rows, G_ * H_), 0) % G_
    col_n = jax.lax.broadcasted_iota(jnp.int32, (rows, G_ * H_), 1) // H_
    mask = (row_n == col_n).astype(jnp.float32)

    for g in range(NG_):
        sl = slice(g * G_, (g + 1) * G_)
        xg = xs_ref[:, sl, :].reshape(rows, MODIN_)
        w1r = w1_ref[sl].reshape(G_ * H_, MODIN_).astype(jnp.bfloat16)
        acc = jax.lax.dot_general(
            xg, w1r, (((1,), (1,)), ((), ())),
            preferred_element_type=jnp.float32)
        b1w = b1_ref[:, g, :]
        hidw = (jnp.tanh(acc + b1w) * mask).astype(jnp.bfloat16)
        w2r = w2_ref[sl].reshape(G_ * H_, MODOUT_).astype(jnp.bfloat16)
        outg = jax.lax.dot_general(
            hidw, w2r, (((1,), (0,)), ((), ())),
            preferred_element_type=jnp.float32)
        outg = outg.reshape(BS_, G_, MODOUT_) + b2_ref[sl][None]
        wconn_ref[:, sl, :] = outg[..., :K_]
        bsc_ref[:, sl, :] = outg[..., K_:K_ + KB_]
        decay_ref[:, sl] = outg[..., K_ + KB_]
        prim_ref[:, sl, :] = outg[..., K_ + KB_ + 1:]

    for c in range(CB_):
        for j in range(BB_):
            idx = bidx_ref[i * CB_ + c, j]
            border_ref[:, c, j:j + 1, :] = bsc_ref[:, pl.ds(c * C_ + idx, 1), :]


def _local_mlp(bidx, htf, hf, dlf, prf, nidf, w1, b1, w2, b2):
    bs = hf.shape[0]
    n_loc = hf.shape[1]
    grid = n_loc // NB_
    nc_loc = n_loc // C_

    grid_spec = pltpu.PrefetchScalarGridSpec(
        num_scalar_prefetch=1,
        grid=(grid,),
        in_specs=[
            pl.BlockSpec((BS_, NB_, K_), lambda i, b: (0, i, 0)),
            pl.BlockSpec((BS_, NB_, D_), lambda i, b: (0, i, 0)),
            pl.BlockSpec((BS_, NB_), lambda i, b: (0, i)),
            pl.BlockSpec((BS_, NB_, D_), lambda i, b: (0, i, 0)),
            pl.BlockSpec((NB_, D_), lambda i, b: (i, 0)),
            pl.BlockSpec((NB_, H_, MODIN_), lambda i, b: (i, 0, 0)),
            pl.BlockSpec((1, NG_, G_ * H_), lambda i, b: (i, 0, 0)),
            pl.BlockSpec((NB_, H_, MODOUT_), lambda i, b: (i, 0, 0)),
            pl.BlockSpec((NB_, MODOUT_), lambda i, b: (i, 0)),
        ],
        out_specs=[
            pl.BlockSpec((BS_, NB_, K_), lambda i, b: (0, i, 0)),
            pl.BlockSpec((BS_, CB_, BB_, KB_), lambda i, b: (0, i, 0, 0)),
            pl.BlockSpec((BS_, NB_), lambda i, b: (0, i)),
            pl.BlockSpec((BS_, NB_, D_), lambda i, b: (0, i, 0)),
        ],
        scratch_shapes=[
            pltpu.VMEM((BS_, NB_, MODIN_), jnp.bfloat16),
            pltpu.VMEM((BS_, NB_, KB_), jnp.float32),
        ],
    )
    return pl.pallas_call(
        _block,
        grid_spec=grid_spec,
        compiler_params=pltpu.CompilerParams(
            dimension_semantics=("arbitrary",)),
        out_shape=[
            jax.ShapeDtypeStruct((bs, n_loc, K_), jnp.float32),
            jax.ShapeDtypeStruct((bs, nc_loc, BB_, KB_), jnp.float32),
            jax.ShapeDtypeStruct((bs, n_loc), jnp.float32),
            jax.ShapeDtypeStruct((bs, n_loc, D_), jnp.float32),
        ],
    )(bidx, htf, hf, dlf, prf, nidf, w1,
      b1.reshape(grid, NG_, G_ * H_), w2, b2)


def kernel(h, hebbian_traces, decay_logit, primitives, mod_w1, mod_b1,
           mod_w2, mod_b2, neuron_id, border_indices):
    bs = h.shape[0]
    htf = hebbian_traces.reshape(bs, N_, K_)
    hf = h.reshape(bs, N_, D_)
    dlf = decay_logit.reshape(bs, N_)
    prf = primitives.reshape(bs, N_, D_)
    nidf = neuron_id.reshape(N_, D_)
    bidx = border_indices.astype(jnp.int32)

    wconn, border, decay, prim = _local_mlp(
        bidx, htf, hf, dlf, prf, nidf, mod_w1, mod_b1, mod_w2, mod_b2)

    return (wconn.reshape(bs, NC_, C_, K_),
            border,
            decay.reshape(bs, NC_, C_),
            prim.reshape(bs, NC_, C_, D_))


# speedup vs baseline: 1.9896x; 1.0010x over previous
import jax
import jax.numpy as jnp
from jax.experimental import pallas as pl
from jax.experimental.pallas import tpu as pltpu
BS_ = 8
NC_ = 16
C_ = 128
D_ = 64
K_ = 16
KB_ = 8
BB_ = 16
H_ = 32
MODIN_ = K_ + 3 * D_ + 1
MODOUT_ = K_ + KB_ + 1 + D_
N_ = NC_ * C_
G_ = 32
NB_ = 256
CB_ = NB_ // C_
NG_ = NB_ // G_


def _block(bidx_ref, ht_ref, h_ref, dl_ref, pr_ref, nid_ref,
           w1_ref, b1_ref, w2_ref, b2_ref,
           wconn_ref, border_ref, decay_ref, prim_ref,
           xs_ref, bsc_ref):
    i = pl.program_id(0)
    rows = G_ * BS_

    xs_ref[:, :, 0:K_] = ht_ref[...].astype(jnp.bfloat16)
    xs_ref[:, :, K_:K_ + D_] = h_ref[...].astype(jnp.bfloat16)
    xs_ref[:, :, K_ + D_:K_ + D_ + 1] = dl_ref[...][..., None].astype(jnp.bfloat16)
    xs_ref[:, :, K_ + D_ + 1:K_ + 2 * D_ + 1] = pr_ref[...].astype(jnp.bfloat16)
    xs_ref[:, :, K_ + 2 * D_ + 1:] = jnp.broadcast_to(
        nid_ref[...].astype(jnp.bfloat16)[None], (BS_, NB_, D_))

    row_n = jax.lax.broadcasted_iota(jnp.int32, (rows, G_ * H_), 0) % G_
    col_n = jax.lax.broadcasted_iota(jnp.int32, (rows, G_ * H_), 1) // H_
    mask = (row_n == col_n).astype(jnp.float32)

    for g in range(NG_):
        if g * G_ % C_ == 0 and g > 0:
            c = g * G_ // C_ - 1
            for j in range(BB_):
                idx = bidx_ref[i * CB_ + c, j]
                border_ref[:, c, j:j + 1, :] = bsc_ref[:, pl.ds(c * C_ + idx, 1), :]
        sl = slice(g * G_, (g + 1) * G_)
        xg = xs_ref[:, sl, :].reshape(rows, MODIN_)
        w1r = w1_ref[sl].reshape(G_ * H_, MODIN_).astype(jnp.bfloat16)
        acc = jax.lax.dot_general(
            xg, w1r, (((1,), (1,)), ((), ())),
            preferred_element_type=jnp.float32)
        b1w = b1_ref[:, g, :]
        hidw = (jnp.tanh(acc + b1w) * mask).astype(jnp.bfloat16)
        w2r = w2_ref[sl].reshape(G_ * H_, MODOUT_).astype(jnp.bfloat16)
        outg = jax.lax.dot_general(
            hidw, w2r, (((1,), (0,)), ((), ())),
            preferred_element_type=jnp.float32)
        outg = outg.reshape(BS_, G_, MODOUT_) + b2_ref[sl][None]
        wconn_ref[:, sl, :] = outg[..., :K_]
        bsc_ref[:, sl, :] = outg[..., K_:K_ + KB_]
        decay_ref[:, sl] = outg[..., K_ + KB_]
        prim_ref[:, sl, :] = outg[..., K_ + KB_ + 1:]

    c = CB_ - 1
    for j in range(BB_):
        idx = bidx_ref[i * CB_ + c, j]
        border_ref[:, c, j:j + 1, :] = bsc_ref[:, pl.ds(c * C_ + idx, 1), :]


def _local_mlp(bidx, htf, hf, dlf, prf, nidf, w1, b1, w2, b2):
    bs = hf.shape[0]
    n_loc = hf.shape[1]
    grid = n_loc // NB_
    nc_loc = n_loc // C_

    grid_spec = pltpu.PrefetchScalarGridSpec(
        num_scalar_prefetch=1,
        grid=(grid,),
        in_specs=[
            pl.BlockSpec((BS_, NB_, K_), lambda i, b: (0, i, 0)),
            pl.BlockSpec((BS_, NB_, D_), lambda i, b: (0, i, 0)),
            pl.BlockSpec((BS_, NB_), lambda i, b: (0, i)),
            pl.BlockSpec((BS_, NB_, D_), lambda i, b: (0, i, 0)),
            pl.BlockSpec((NB_, D_), lambda i, b: (i, 0)),
            pl.BlockSpec((NB_, H_, MODIN_), lambda i, b: (i, 0, 0)),
            pl.BlockSpec((1, NG_, G_ * H_), lambda i, b: (i, 0, 0)),
            pl.BlockSpec((NB_, H_, MODOUT_), lambda i, b: (i, 0, 0)),
            pl.BlockSpec((NB_, MODOUT_), lambda i, b: (i, 0)),
        ],
        out_specs=[
            pl.BlockSpec((BS_, NB_, K_), lambda i, b: (0, i, 0)),
            pl.BlockSpec((BS_, CB_, BB_, KB_), lambda i, b: (0, i, 0, 0)),
            pl.BlockSpec((BS_, NB_), lambda i, b: (0, i)),
            pl.BlockSpec((BS_, NB_, D_), lambda i, b: (0, i, 0)),
        ],
        scratch_shapes=[
            pltpu.VMEM((BS_, NB_, MODIN_), jnp.bfloat16),
            pltpu.VMEM((BS_, NB_, KB_), jnp.float32),
        ],
    )
    return pl.pallas_call(
        _block,
        grid_spec=grid_spec,
        compiler_params=pltpu.CompilerParams(
            dimension_semantics=("arbitrary",)),
        out_shape=[
            jax.ShapeDtypeStruct((bs, n_loc, K_), jnp.float32),
            jax.ShapeDtypeStruct((bs, nc_loc, BB_, KB_), jnp.float32),
            jax.ShapeDtypeStruct((bs, n_loc), jnp.float32),
            jax.ShapeDtypeStruct((bs, n_loc, D_), jnp.float32),
        ],
    )(bidx, htf, hf, dlf, prf, nidf, w1,
      b1.reshape(grid, NG_, G_ * H_), w2, b2)


def kernel(h, hebbian_traces, decay_logit, primitives, mod_w1, mod_b1,
           mod_w2, mod_b2, neuron_id, border_indices):
    bs = h.shape[0]
    htf = hebbian_traces.reshape(bs, N_, K_)
    hf = h.reshape(bs, N_, D_)
    dlf = decay_logit.reshape(bs, N_)
    prf = primitives.reshape(bs, N_, D_)
    nidf = neuron_id.reshape(N_, D_)
    bidx = border_indices.astype(jnp.int32)

    wconn, border, decay, prim = _local_mlp(
        bidx, htf, hf, dlf, prf, nidf, mod_w1, mod_b1, mod_w2, mod_b2)

    return (wconn.reshape(bs, NC_, C_, K_),
            border,
            decay.reshape(bs, NC_, C_),
            prim.reshape(bs, NC_, C_, D_))
